# Initial kernel scaffold; baseline (speedup 1.0000x reference)
import jax
import jax.numpy as jnp
from jax.experimental import pallas as pl


def kernel(x, pos_table, lum_table, W, b):
    raise NotImplementedError("write your pallas kernel here")


# speedup vs baseline: 1.6121x; 1.6121x over previous
import functools

import jax
import jax.numpy as jnp
from jax.experimental import pallas as pl
from jax.experimental.pallas import tpu as pltpu

_B = 8
_P = 784
_NLEV = 1000
_D = 10000
_DPAD = 10240
_TD = 2048
_SUB = _TD // 128
_GRID = _DPAD // _TD
_NCLS = 10


def _encode_body(xs_ref, lum_ref, pos_ref, out_ref):
    for b in range(_B):
        base = b * _P

        def body(i, acc):
            for j in range(8):
                p = i * 8 + j
                idx = xs_ref[base + p]
                acc = acc + lum_ref[idx] * pos_ref[p]
            return acc

        acc = jax.lax.fori_loop(
            0, _P // 8, body, jnp.zeros((_SUB, 128), jnp.float32))
        out_ref[b] = jnp.tanh(acc)


def _classify_body(enc_ref, w_ref, bias_ref, out_ref):
    out_ref[...] = jax.lax.dot_general(
        enc_ref[...], w_ref[...],
        (((1,), (1,)), ((), ())),
        preferred_element_type=jnp.float32) + bias_ref[...]


def kernel(x, pos_table, lum_table, W, b):
    xf = x.reshape(_B * _P).astype(jnp.int32)
    lum3 = jnp.pad(lum_table, ((0, 0), (0, _DPAD - _D))).reshape(
        _NLEV, _DPAD // 128, 128)
    pos3 = jnp.pad(pos_table, ((0, 0), (0, _DPAD - _D))).reshape(
        _P, _DPAD // 128, 128)
    enc3 = pl.pallas_call(
        _encode_body,
        grid_spec=pltpu.PrefetchScalarGridSpec(
            num_scalar_prefetch=1,
            grid=(_GRID,),
            in_specs=[
                pl.BlockSpec((_NLEV, _SUB, 128), lambda d, xs: (0, d, 0)),
                pl.BlockSpec((_P, _SUB, 128), lambda d, xs: (0, d, 0)),
            ],
            out_specs=pl.BlockSpec((_B, _SUB, 128), lambda d, xs: (0, d, 0)),
        ),
        out_shape=jax.ShapeDtypeStruct((_B, _DPAD // 128, 128), jnp.float32),
    )(xf, lum3, pos3)
    enc = enc3.reshape(_B, _DPAD)
    w_pad = jnp.pad(W, ((0, 128 - _NCLS), (0, _DPAD - _D)))
    bias_pad = jnp.pad(b, (0, 128 - _NCLS)).reshape(1, 128)
    logits = pl.pallas_call(
        _classify_body,
        out_shape=jax.ShapeDtypeStruct((_B, 128), jnp.float32),
    )(enc, w_pad, bias_pad)
    return logits[:, :_NCLS]


# speedup vs baseline: 3.2545x; 2.0188x over previous
import functools

import jax
import jax.numpy as jnp
from jax.experimental import pallas as pl
from jax.experimental.pallas import tpu as pltpu

_B = 8
_P = 784
_NLEV = 1000
_D = 10000
_DPAD = 10240
_TD = 2048
_SUB = _TD // 128
_GRID = _DPAD // _TD
_NCLS = 10
_FRAG = _D % 128


def _issue_copies(dd, slot, lum_hbm, pos_hbm, lum_t, pos_t, lum_v, pos_v,
                  sem_l, sem_p, start):
    for hbm, tail, vbuf, sem, nrows in (
            (lum_hbm, lum_t, lum_v, sem_l, _NLEV),
            (pos_hbm, pos_t, pos_v, sem_p, _P)):
        for s in range(_SUB):
            col = dd * _TD + s * 128
            full = col + 128 <= _D
            frag = jnp.logical_and(col < _D, col + 128 > _D)
            cp_full = pltpu.make_async_copy(
                hbm.at[:, pl.ds(col, 128)], vbuf.at[slot, :, s, :], sem)
            cp_frag = pltpu.make_async_copy(
                tail, vbuf.at[slot, :, s, :], sem)

            @pl.when(full)
            def _():
                if start:
                    cp_full.start()
                else:
                    cp_full.wait()

            @pl.when(frag)
            def _():
                if start:
                    cp_frag.start()
                else:
                    cp_frag.wait()
    if start:
        @pl.when(dd == _GRID - 1)
        def _():
            lum_v[slot, :, _SUB - 1, :] = jnp.zeros((_NLEV, 128), jnp.float32)
            pos_v[slot, :, _SUB - 1, :] = jnp.zeros((_P, 128), jnp.float32)


def _encode_body(xs_ref, lum_hbm, pos_hbm, lum_t, pos_t, out_ref, lum_v,
                 pos_v, sem_l0, sem_l1, sem_p0, sem_p1):
    d = pl.program_id(0)
    args = (lum_hbm, pos_hbm, lum_t, pos_t, lum_v, pos_v)

    @pl.when(d == 0)
    def _():
        _issue_copies(0, 0, *args, sem_l0, sem_p0, start=True)

    @pl.when(d % 2 == 0)
    def _():
        _issue_copies(d, 0, *args, sem_l0, sem_p0, start=False)

    @pl.when(d % 2 == 1)
    def _():
        _issue_copies(d, 1, *args, sem_l1, sem_p1, start=False)

    @pl.when(jnp.logical_and(d + 1 < _GRID, (d + 1) % 2 == 0))
    def _():
        _issue_copies(d + 1, 0, *args, sem_l0, sem_p0, start=True)

    @pl.when(jnp.logical_and(d + 1 < _GRID, (d + 1) % 2 == 1))
    def _():
        _issue_copies(d + 1, 1, *args, sem_l1, sem_p1, start=True)

    slot = d % 2

    def body(p, accs):
        pos_row = pos_v[slot, p]
        return tuple(
            accs[b] + lum_v[slot, xs_ref[b * _P + p]] * pos_row
            for b in range(_B))

    accs = jax.lax.fori_loop(
        0, _P, body,
        tuple(jnp.zeros((_SUB, 128), jnp.float32) for _ in range(_B)))
    for b in range(_B):
        out_ref[b] = jnp.tanh(accs[b])


def _classify_body(enc_ref, w_ref, bias_ref, out_ref):
    out_ref[...] = jax.lax.dot_general(
        enc_ref[...], w_ref[...],
        (((1,), (1,)), ((), ())),
        preferred_element_type=jnp.float32) + bias_ref[...]


def kernel(x, pos_table, lum_table, W, b):
    xf = x.reshape(_B * _P).astype(jnp.int32)
    ncut = (_D // 128) * 128
    lum_tail = jnp.pad(lum_table[:, ncut:], ((0, 0), (0, 128 - _FRAG)))
    pos_tail = jnp.pad(pos_table[:, ncut:], ((0, 0), (0, 128 - _FRAG)))
    enc3 = pl.pallas_call(
        _encode_body,
        grid_spec=pltpu.PrefetchScalarGridSpec(
            num_scalar_prefetch=1,
            grid=(_GRID,),
            in_specs=[
                pl.BlockSpec(memory_space=pl.ANY),
                pl.BlockSpec(memory_space=pl.ANY),
                pl.BlockSpec(memory_space=pl.ANY),
                pl.BlockSpec(memory_space=pl.ANY),
            ],
            out_specs=pl.BlockSpec((_B, _SUB, 128), lambda d, xs: (0, d, 0)),
            scratch_shapes=[
                pltpu.VMEM((2, _NLEV, _SUB, 128), jnp.float32),
                pltpu.VMEM((2, _P, _SUB, 128), jnp.float32),
                pltpu.SemaphoreType.DMA,
                pltpu.SemaphoreType.DMA,
                pltpu.SemaphoreType.DMA,
                pltpu.SemaphoreType.DMA,
            ],
        ),
        out_shape=jax.ShapeDtypeStruct((_B, _DPAD // 128, 128), jnp.float32),
    )(xf, lum_table, pos_table, lum_tail, pos_tail)
    enc = enc3.reshape(_B, _DPAD)
    w_pad = jnp.pad(W, ((0, 128 - _NCLS), (0, _DPAD - _D)))
    bias_pad = jnp.pad(b, (0, 128 - _NCLS)).reshape(1, 128)
    logits = pl.pallas_call(
        _classify_body,
        out_shape=jax.ShapeDtypeStruct((_B, 128), jnp.float32),
    )(enc, w_pad, bias_pad)
    return logits[:, :_NCLS]
